# speedup vs baseline: 12.2104x; 12.2104x over previous
import functools

import jax
import jax.numpy as jnp
from jax import lax
from jax.experimental import pallas as pl
from jax.experimental.pallas import tpu as pltpu
from jax.experimental.pallas import tpu_sc as plsc

B = 1024
T = 200
VOCAB = 100000
N_EMBD = 128
COUNT_DIM = 8
NUM_PROPS = 520
FP_DIM = NUM_PROPS - COUNT_DIM

NC, NS = 2, 16
NW = NC * NS
ROWS = B * T
RPW = ROWS // NW
CH = 640


def _sc_gather(table, idx):
    mesh = plsc.VectorSubcoreMesh(core_axis_name="c", subcore_axis_name="s")

    @functools.partial(
        pl.kernel,
        mesh=mesh,
        out_type=jax.ShapeDtypeStruct((ROWS, N_EMBD), jnp.float32),
        scratch_types=[
            pltpu.VMEM((CH,), jnp.int32),
            pltpu.VMEM((CH, N_EMBD), jnp.float32),
            pltpu.SemaphoreType.DMA,
        ],
    )
    def k(table_hbm, idx_hbm, out_hbm, idx_v, rows_v, sem):
        wid = lax.axis_index("s") * NC + lax.axis_index("c")
        for i in range(RPW // CH):
            base = wid * RPW + i * CH
            pltpu.sync_copy(idx_hbm.at[pl.ds(base, CH)], idx_v)
            pltpu.async_copy(table_hbm.at[idx_v], rows_v, sem).wait()
            pltpu.sync_copy(rows_v, out_hbm.at[pl.ds(base, CH)])

    return k(table, idx)


BB = 16


def _tc_body(gath_ref, prop_ref, base_ref, diff_ref, cos_ref, sin_ref,
             tt1_ref, out_ref):
    propf = prop_ref[...].astype(jnp.float32)
    pemb = base_ref[...][None] + propf[:, :, None] * diff_ref[...][None]
    out_ref[:, :NUM_PROPS, :] = pemb
    g = gath_ref[...]
    h = N_EMBD // 2
    rh = jnp.concatenate([-g[..., h:], g[..., :h]], axis=-1)
    out_ref[:, NUM_PROPS:, :] = (g * cos_ref[...][None]
                                 + rh * sin_ref[...][None]
                                 + tt1_ref[...][None])


def kernel(token, prop, tok_table, type_table, prop_type_table, cnt_bit,
           cnt_val, fp_pair, fp_bit, fp_val):
    idx = token.reshape(ROWS).astype(jnp.int32)
    gathered = _sc_gather(tok_table, idx).reshape(B, T, N_EMBD)

    inv_freq = 1.0 / (10000.0 ** (jnp.arange(0, N_EMBD, 2, dtype=jnp.float32)
                                  / N_EMBD))
    freqs = jnp.arange(T, dtype=jnp.float32)[:, None] * inv_freq[None, :]
    pos = jnp.concatenate([freqs, freqs], axis=-1)
    cos, sin = jnp.cos(pos), jnp.sin(pos)

    base_cnt = cnt_val[0][None] + cnt_bit + prop_type_table[0][None]
    pair_rep = jnp.repeat(fp_pair, 2, axis=0)
    bit_rep = jnp.tile(fp_bit, (FP_DIM // 2, 1))
    base_fp = fp_val[0][None] + pair_rep + bit_rep + prop_type_table[1][None]
    base = jnp.concatenate([base_cnt, base_fp], axis=0) + type_table[0][None]
    diff = jnp.concatenate([
        jnp.broadcast_to(cnt_val[1] - cnt_val[0], (COUNT_DIM, N_EMBD)),
        jnp.broadcast_to(fp_val[1] - fp_val[0], (FP_DIM, N_EMBD)),
    ], axis=0)
    tt1 = type_table[1][None]

    grid = (B // BB,)
    return pl.pallas_call(
        _tc_body,
        grid=grid,
        in_specs=[
            pl.BlockSpec((BB, T, N_EMBD), lambda i: (i, 0, 0)),
            pl.BlockSpec((BB, NUM_PROPS), lambda i: (i, 0)),
            pl.BlockSpec((NUM_PROPS, N_EMBD), lambda i: (0, 0)),
            pl.BlockSpec((NUM_PROPS, N_EMBD), lambda i: (0, 0)),
            pl.BlockSpec((T, N_EMBD), lambda i: (0, 0)),
            pl.BlockSpec((T, N_EMBD), lambda i: (0, 0)),
            pl.BlockSpec((1, N_EMBD), lambda i: (0, 0)),
        ],
        out_specs=pl.BlockSpec((BB, NUM_PROPS + T, N_EMBD), lambda i: (i, 0, 0)),
        out_shape=jax.ShapeDtypeStruct((B, NUM_PROPS + T, N_EMBD), jnp.float32),
    )(gathered, prop, base, diff, cos, sin, tt1)
